# baseline (speedup 1.0000x reference)
import functools

import jax
import jax.numpy as jnp
from jax import lax
from jax.experimental import pallas as pl
from jax.experimental.pallas import tpu as pltpu, tpu_sc as plsc

_NC = 2
_NS = 16
_NW = _NC * _NS
_LANES = 16
_IDXW = 128


def _lane_perm(x, p):
    return lax.gather(
        x, p[:, None],
        lax.GatherDimensionNumbers(
            offset_dims=(), collapsed_slice_dims=(0,), start_index_map=(0,)),
        slice_sizes=(1,),
        mode=lax.GatherScatterMode.PROMISE_IN_BOUNDS)


def _build_sc_call(B, C, V, E):
    b_per_w = B // _NW
    CB = 128
    NCH = b_per_w // CB
    PB = CB * C
    TR = b_per_w // _IDXW
    CR = (b_per_w * C) // _IDXW
    CPC = CR // NCH
    EV = E // _LANES

    mesh = plsc.VectorSubcoreMesh(core_axis_name="c", subcore_axis_name="s")

    @functools.partial(
        pl.kernel,
        out_type=jax.ShapeDtypeStruct((B * C,), jnp.float32),
        mesh=mesh,
        compiler_params=pltpu.CompilerParams(use_tc_tiling_on_sc=False),
        scratch_types=[
            pltpu.VMEM((TR, _IDXW), jnp.int32),
            pltpu.VMEM((CR, _IDXW), jnp.int32),
            pltpu.VMEM((CB, E), jnp.float32),
            pltpu.VMEM((PB, E), jnp.float32),
            pltpu.VMEM((PB,), jnp.float32),
            pltpu.SemaphoreType.DMA,
        ],
    )
    def sc_call(tgt_hbm, ctx_hbm, ttab_hbm, ctab_hbm, out_hbm,
                tidx_v, cidx_v, trows_v, crows_v, outv, sem):
        wid = lax.axis_index("s") * _NC + lax.axis_index("c")
        staged = [
            pltpu.async_copy(
                tgt_hbm.at[pl.ds((wid * TR + j) * _IDXW, _IDXW)],
                tidx_v.at[j], sem)
            for j in range(TR)
        ] + [
            pltpu.async_copy(
                ctx_hbm.at[pl.ds((wid * CR + j) * _IDXW, _IDXW)],
                cidx_v.at[j], sem)
            for j in range(CR)
        ]
        for s in staged:
            s.wait()

        for k in range(NCH):
            started = [pltpu.async_copy(ttab_hbm.at[tidx_v.at[k]], trows_v, sem)]
            for j in range(CPC):
                started.append(pltpu.async_copy(
                    ctab_hbm.at[cidx_v.at[k * CPC + j]],
                    crows_v.at[pl.ds(j * _IDXW, _IDXW)], sem))
            for s in started:
                s.wait()

            lanes = lax.iota(jnp.int32, 16)
            perms = [lanes ^ sh for sh in (8, 4, 2, 1)]

            def gbody(g, carry):
                tb = g * _LANES
                pb = g * _LANES * C
                for j in range(C):
                    trow = {}
                    dotv = jnp.zeros((_LANES,), jnp.float32)
                    for i in range(_LANES):
                        q = j * _LANES + i
                        bi, _ = divmod(q, C)
                        if bi not in trow:
                            trow[bi] = [
                                trows_v[tb + bi, pl.ds(v * _LANES, _LANES)]
                                for v in range(EV)]
                        tr = trow[bi]
                        acc = tr[0] * crows_v[pb + q, pl.ds(0, _LANES)]
                        for v in range(1, EV):
                            acc = acc + tr[v] * crows_v[pb + q,
                                                        pl.ds(v * _LANES, _LANES)]
                        for pm in perms:
                            acc = acc + _lane_perm(acc, pm)
                        dotv = jnp.where(lanes == i, acc, dotv)
                    outv[pl.ds(pb + j * _LANES, _LANES)] = dotv
                return carry

            lax.fori_loop(0, CB // _LANES, gbody, 0)
            pltpu.sync_copy(
                outv, out_hbm.at[pl.ds((wid * NCH + k) * PB, PB)])

    return sc_call


def kernel(target, context, target_table, context_table):
    if target.ndim == 2:
        target = jnp.squeeze(target, axis=1)
    B = target.shape[0]
    C = context.shape[1]
    V, E = target_table.shape
    sc_call = _build_sc_call(B, C, V, E)
    tgt1 = target.astype(jnp.int32).reshape(B)
    ctx1 = context.astype(jnp.int32).reshape(B * C)
    out_flat = sc_call(tgt1, ctx1, target_table, context_table)
    return out_flat.reshape(B, C)
